# baseline (speedup 1.0000x reference)
import functools

import jax
import jax.numpy as jnp
from jax import lax
from jax.experimental import pallas as pl
from jax.experimental.pallas import tpu as pltpu

K = 16
N = 16
P = 2048
C = 64
EDGES = N * P * K
RT = 256
IMAX = 2147483647


def _knn_kernel(pts_r_ref, ptsT_ref, out_ref):
    n = pl.program_id(0)
    pr = pts_r_ref[0]
    pcT = ptsT_ref[0]
    rr = jnp.sum(pr * pr, axis=1, keepdims=True)
    rc = jnp.sum(pcT * pcT, axis=0, keepdims=True)
    dot = jnp.dot(pr.astype(jnp.bfloat16), pcT.astype(jnp.bfloat16),
                  preferred_element_type=jnp.float32)
    dist = rr - 2.0 * dot + rc
    col = lax.broadcasted_iota(jnp.int32, (RT, P), 1)
    bits = lax.bitcast_convert_type(dist, jnp.int32)
    okey = jnp.where(bits >= 0, bits, jnp.bitwise_xor(bits, jnp.int32(0x7FFFFFFF)))
    keys0 = jnp.bitwise_or(jnp.bitwise_and(okey, jnp.int32(-2048)), col)

    def body(k, carry):
        keys, acc = carry
        m = keys[:, 0:128]
        for j in range(1, 16):
            m = jnp.minimum(m, keys[:, j * 128:(j + 1) * 128])
        g = jnp.min(m, axis=1, keepdims=True)
        keys = jnp.where(keys == g, IMAX, keys)
        lane = lax.broadcasted_iota(jnp.int32, (RT, K + 1), 1)
        acc = jnp.where(lane == k, jnp.bitwise_and(g, jnp.int32(2047)), acc)
        return keys, acc

    _, acc = lax.fori_loop(0, K + 1, body,
                           (keys0, jnp.zeros((RT, K + 1), jnp.int32)))
    out_ref[0] = acc[:, 1:] + n * P


def _knn(pts):
    ptsT = jnp.transpose(pts, (0, 2, 1))
    return pl.pallas_call(
        _knn_kernel,
        grid=(N, P // RT),
        in_specs=[
            pl.BlockSpec((1, RT, 2), lambda n, r: (n, r, 0)),
            pl.BlockSpec((1, 2, P), lambda n, r: (n, 0, 0)),
        ],
        out_specs=pl.BlockSpec((1, RT, K), lambda n, r: (n, r, 0)),
        out_shape=jax.ShapeDtypeStruct((N, P, K), jnp.int32),
    )(pts, ptsT)


def _stats_kernel(x_ref, s_ref, ss_ref):
    i = pl.program_id(0)
    x = x_ref[...]
    s = jnp.sum(x, axis=0, keepdims=True)
    ss = jnp.sum(x * x, axis=0, keepdims=True)

    @pl.when(i == 0)
    def _():
        s_ref[...] = s
        ss_ref[...] = ss

    @pl.when(i > 0)
    def _():
        s_ref[...] += s
        ss_ref[...] += ss


def _stats(x, rows_per_block):
    rows = x.shape[0]
    grid = rows // rows_per_block
    s, ss = pl.pallas_call(
        _stats_kernel,
        grid=(grid,),
        in_specs=[pl.BlockSpec((rows_per_block, 64), lambda i: (i, 0))],
        out_specs=[
            pl.BlockSpec((1, 64), lambda i: (0, 0)),
            pl.BlockSpec((1, 64), lambda i: (0, 0)),
        ],
        out_shape=[
            jax.ShapeDtypeStruct((1, 64), jnp.float32),
            jax.ShapeDtypeStruct((1, 64), jnp.float32),
        ],
    )(x)
    return s[0], ss[0]


def _mid_kernel(y0_ref, a0_ref, c0_ref, w1_ref, y1_ref, s_ref, ss_ref):
    i = pl.program_id(0)
    e0 = jnp.maximum(y0_ref[...] * a0_ref[...] + c0_ref[...], 0.0)
    y1 = jnp.dot(e0, w1_ref[...], preferred_element_type=jnp.float32)
    y1_ref[...] = y1
    s = jnp.sum(y1, axis=0, keepdims=True)
    ss = jnp.sum(y1 * y1, axis=0, keepdims=True)

    @pl.when(i == 0)
    def _():
        s_ref[...] = s
        ss_ref[...] = ss

    @pl.when(i > 0)
    def _():
        s_ref[...] += s
        ss_ref[...] += ss


def _mid(y0, a0, c0, w1t, rows_per_block):
    rows = y0.shape[0]
    grid = rows // rows_per_block
    y1, s, ss = pl.pallas_call(
        _mid_kernel,
        grid=(grid,),
        in_specs=[
            pl.BlockSpec((rows_per_block, 64), lambda i: (i, 0)),
            pl.BlockSpec((1, 64), lambda i: (0, 0)),
            pl.BlockSpec((1, 64), lambda i: (0, 0)),
            pl.BlockSpec((64, 64), lambda i: (0, 0)),
        ],
        out_specs=[
            pl.BlockSpec((rows_per_block, 64), lambda i: (i, 0)),
            pl.BlockSpec((1, 64), lambda i: (0, 0)),
            pl.BlockSpec((1, 64), lambda i: (0, 0)),
        ],
        out_shape=[
            jax.ShapeDtypeStruct((rows, 64), jnp.float32),
            jax.ShapeDtypeStruct((1, 64), jnp.float32),
            jax.ShapeDtypeStruct((1, 64), jnp.float32),
        ],
    )(y0, a0[None, :], c0[None, :], w1t)
    return y1, s[0], ss[0]


def _final_kernel(y1_ref, a1_ref, c1_ref, sc_ref, out_ref):
    e1 = jnp.maximum(y1_ref[...] * a1_ref[...] + c1_ref[...], 0.0)
    rows = e1.shape[0]
    h = jnp.mean(e1.reshape(rows // K, K, 64), axis=1)
    out_ref[...] = jnp.maximum(h + sc_ref[...], 0.0)


def _final(y1, a1, c1, sc, rows_per_block):
    rows = y1.shape[0]
    grid = rows // rows_per_block
    out = pl.pallas_call(
        _final_kernel,
        grid=(grid,),
        in_specs=[
            pl.BlockSpec((rows_per_block, 64), lambda i: (i, 0)),
            pl.BlockSpec((1, 64), lambda i: (0, 0)),
            pl.BlockSpec((1, 64), lambda i: (0, 0)),
            pl.BlockSpec((rows_per_block // K, 64), lambda i: (i, 0)),
        ],
        out_specs=pl.BlockSpec((rows_per_block // K, 64), lambda i: (i, 0)),
        out_shape=jax.ShapeDtypeStruct((rows // K, 64), jnp.float32),
    )(y1, a1[None, :], c1[None, :], sc)
    return out


def _bn_coeffs(s, ss, count, g, be):
    m = s / count
    v = ss / count - m * m
    a = g * lax.rsqrt(v + 1e-5)
    c = be - m * a
    return a, c


def kernel(features, W0, b0, g0, be0, W1, b1, g1, be1, Ws, bs, gs, bes):
    pts = features[:, :, 0:2]
    X = features[:, :, 2:]

    gidx = _knn(pts).reshape(EDGES)

    W0a = W0[:, :C]
    W0b = W0[:, C:]
    U = jnp.einsum('npc,oc->npo', X, W0b).reshape(N * P, C)
    V = jnp.einsum('npc,oc->npo', X, W0a + W0b) + b0[None, None, :]

    G = jnp.take_along_axis(U, gidx[:, None], axis=0)
    y0 = (jnp.repeat(V.reshape(N * P, C), K, axis=0) - G)

    s0, ss0 = _stats(y0, 8192)
    a0, c0 = _bn_coeffs(s0, ss0, float(EDGES), g0, be0)
    y1, s1, ss1 = _mid(y0, a0, c0, W1.T, 8192)
    a1, c1 = _bn_coeffs(s1, ss1, float(EDGES), g1, be1)

    ysc = (jnp.einsum('npc,oc->npo', X, Ws) + bs[None, None, :]).reshape(N * P, C)
    ssc, sssc = _stats(ysc, 4096)
    asc, csc = _bn_coeffs(ssc, sssc, float(N * P), gs, bes)
    sc = ysc * asc[None, :] + csc[None, :]

    out = _final(y1, a1, c1, sc, 8192)
    return jnp.transpose(out.reshape(N, P, C), (0, 2, 1))


# speedup vs baseline: 1.4800x; 1.4800x over previous
import functools

import jax
import jax.numpy as jnp
from jax import lax
from jax.experimental import pallas as pl
from jax.experimental.pallas import tpu as pltpu
from jax.experimental.pallas import tpu_sc as plsc

K = 16
N = 16
P = 2048
C = 64
EDGES = N * P * K
RT = 256
IMAX = 2147483647


def _knn_kernel(pts_r_ref, ptsT_ref, out_ref):
    n = pl.program_id(0)
    pr = pts_r_ref[0]
    pcT = ptsT_ref[0]
    rr = jnp.sum(pr * pr, axis=1, keepdims=True)
    rc = jnp.sum(pcT * pcT, axis=0, keepdims=True)
    dot = jnp.dot(pr.astype(jnp.bfloat16), pcT.astype(jnp.bfloat16),
                  preferred_element_type=jnp.float32)
    dist = rr - 2.0 * dot + rc
    col = lax.broadcasted_iota(jnp.int32, (RT, P), 1)
    bits = lax.bitcast_convert_type(dist, jnp.int32)
    okey = jnp.where(bits >= 0, bits, jnp.bitwise_xor(bits, jnp.int32(0x7FFFFFFF)))
    keys0 = jnp.bitwise_or(jnp.bitwise_and(okey, jnp.int32(-2048)), col)

    def body(k, carry):
        keys, acc = carry
        m = keys[:, 0:128]
        for j in range(1, 16):
            m = jnp.minimum(m, keys[:, j * 128:(j + 1) * 128])
        g = jnp.min(m, axis=1, keepdims=True)
        keys = jnp.where(keys == g, IMAX, keys)
        lane = lax.broadcasted_iota(jnp.int32, (RT, K + 1), 1)
        acc = jnp.where(lane == k, jnp.bitwise_and(g, jnp.int32(2047)), acc)
        return keys, acc

    _, acc = lax.fori_loop(0, K + 1, body,
                           (keys0, jnp.zeros((RT, K + 1), jnp.int32)))
    out_ref[0] = acc[:, 1:] + n * P


def _knn(pts):
    ptsT = jnp.transpose(pts, (0, 2, 1))
    return pl.pallas_call(
        _knn_kernel,
        grid=(N, P // RT),
        in_specs=[
            pl.BlockSpec((1, RT, 2), lambda n, r: (n, r, 0)),
            pl.BlockSpec((1, 2, P), lambda n, r: (n, 0, 0)),
        ],
        out_specs=pl.BlockSpec((1, RT, K), lambda n, r: (n, r, 0)),
        out_shape=jax.ShapeDtypeStruct((N, P, K), jnp.int32),
    )(pts, ptsT)


NW = 32
CH = 1024
EPW = EDGES // NW


def _sc_gather(table, idx):
    mesh = plsc.VectorSubcoreMesh(core_axis_name="c", subcore_axis_name="s")

    @functools.partial(
        pl.kernel, mesh=mesh,
        out_type=jax.ShapeDtypeStruct((EDGES, C), jnp.float32),
        compiler_params=pltpu.CompilerParams(use_tc_tiling_on_sc=False),
        scratch_types=[
            pltpu.VMEM((CH,), jnp.int32),
            pltpu.VMEM((CH, C), jnp.float32),
            pltpu.SemaphoreType.DMA,
        ],
    )
    def k(table_hbm, idx_hbm, out_hbm, idx_v, rows_v, sem):
        wid = lax.axis_index("s") * 2 + lax.axis_index("c")
        base = wid * EPW

        def body(j, carry):
            off = base + j * CH
            pltpu.sync_copy(idx_hbm.at[pl.ds(off, CH)], idx_v)
            pltpu.async_copy(table_hbm.at[idx_v], rows_v, sem).wait()
            pltpu.sync_copy(rows_v, out_hbm.at[pl.ds(off, CH)])
            return carry

        lax.fori_loop(0, EPW // CH, body, 0)

    return k(table, idx)


def _stats_kernel(x_ref, s_ref, ss_ref):
    i = pl.program_id(0)
    x = x_ref[...]
    s = jnp.sum(x, axis=0, keepdims=True)
    ss = jnp.sum(x * x, axis=0, keepdims=True)

    @pl.when(i == 0)
    def _():
        s_ref[...] = s
        ss_ref[...] = ss

    @pl.when(i > 0)
    def _():
        s_ref[...] += s
        ss_ref[...] += ss


def _stats(x, rows_per_block):
    rows = x.shape[0]
    grid = rows // rows_per_block
    s, ss = pl.pallas_call(
        _stats_kernel,
        grid=(grid,),
        in_specs=[pl.BlockSpec((rows_per_block, 64), lambda i: (i, 0))],
        out_specs=[
            pl.BlockSpec((1, 64), lambda i: (0, 0)),
            pl.BlockSpec((1, 64), lambda i: (0, 0)),
        ],
        out_shape=[
            jax.ShapeDtypeStruct((1, 64), jnp.float32),
            jax.ShapeDtypeStruct((1, 64), jnp.float32),
        ],
    )(x)
    return s[0], ss[0]


def _mid_kernel(y0_ref, a0_ref, c0_ref, w1_ref, y1_ref, s_ref, ss_ref):
    i = pl.program_id(0)
    e0 = jnp.maximum(y0_ref[...] * a0_ref[...] + c0_ref[...], 0.0)
    y1 = jnp.dot(e0, w1_ref[...], preferred_element_type=jnp.float32)
    y1_ref[...] = y1
    s = jnp.sum(y1, axis=0, keepdims=True)
    ss = jnp.sum(y1 * y1, axis=0, keepdims=True)

    @pl.when(i == 0)
    def _():
        s_ref[...] = s
        ss_ref[...] = ss

    @pl.when(i > 0)
    def _():
        s_ref[...] += s
        ss_ref[...] += ss


def _mid(y0, a0, c0, w1t, rows_per_block):
    rows = y0.shape[0]
    grid = rows // rows_per_block
    y1, s, ss = pl.pallas_call(
        _mid_kernel,
        grid=(grid,),
        in_specs=[
            pl.BlockSpec((rows_per_block, 64), lambda i: (i, 0)),
            pl.BlockSpec((1, 64), lambda i: (0, 0)),
            pl.BlockSpec((1, 64), lambda i: (0, 0)),
            pl.BlockSpec((64, 64), lambda i: (0, 0)),
        ],
        out_specs=[
            pl.BlockSpec((rows_per_block, 64), lambda i: (i, 0)),
            pl.BlockSpec((1, 64), lambda i: (0, 0)),
            pl.BlockSpec((1, 64), lambda i: (0, 0)),
        ],
        out_shape=[
            jax.ShapeDtypeStruct((rows, 64), jnp.float32),
            jax.ShapeDtypeStruct((1, 64), jnp.float32),
            jax.ShapeDtypeStruct((1, 64), jnp.float32),
        ],
    )(y0, a0[None, :], c0[None, :], w1t)
    return y1, s[0], ss[0]


def _final_kernel(y1_ref, a1_ref, c1_ref, sc_ref, out_ref):
    e1 = jnp.maximum(y1_ref[...] * a1_ref[...] + c1_ref[...], 0.0)
    rows = e1.shape[0]
    h = jnp.mean(e1.reshape(rows // K, K, 64), axis=1)
    out_ref[...] = jnp.maximum(h + sc_ref[...], 0.0)


def _final(y1, a1, c1, sc, rows_per_block):
    rows = y1.shape[0]
    grid = rows // rows_per_block
    out = pl.pallas_call(
        _final_kernel,
        grid=(grid,),
        in_specs=[
            pl.BlockSpec((rows_per_block, 64), lambda i: (i, 0)),
            pl.BlockSpec((1, 64), lambda i: (0, 0)),
            pl.BlockSpec((1, 64), lambda i: (0, 0)),
            pl.BlockSpec((rows_per_block // K, 64), lambda i: (i, 0)),
        ],
        out_specs=pl.BlockSpec((rows_per_block // K, 64), lambda i: (i, 0)),
        out_shape=jax.ShapeDtypeStruct((rows // K, 64), jnp.float32),
    )(y1, a1[None, :], c1[None, :], sc)
    return out


def _bn_coeffs(s, ss, count, g, be):
    m = s / count
    v = ss / count - m * m
    a = g * lax.rsqrt(v + 1e-5)
    c = be - m * a
    return a, c


def kernel(features, W0, b0, g0, be0, W1, b1, g1, be1, Ws, bs, gs, bes):
    pts = features[:, :, 0:2]
    X = features[:, :, 2:]

    gidx = _knn(pts).reshape(EDGES)

    W0a = W0[:, :C]
    W0b = W0[:, C:]
    U = jnp.einsum('npc,oc->npo', X, W0b).reshape(N * P, C)
    V = jnp.einsum('npc,oc->npo', X, W0a + W0b) + b0[None, None, :]

    G = _sc_gather(U, gidx)
    y0 = (jnp.repeat(V.reshape(N * P, C), K, axis=0) - G)

    s0, ss0 = _stats(y0, 8192)
    a0, c0 = _bn_coeffs(s0, ss0, float(EDGES), g0, be0)
    y1, s1, ss1 = _mid(y0, a0, c0, W1.T, 8192)
    a1, c1 = _bn_coeffs(s1, ss1, float(EDGES), g1, be1)

    ysc = (jnp.einsum('npc,oc->npo', X, Ws) + bs[None, None, :]).reshape(N * P, C)
    ssc, sssc = _stats(ysc, 4096)
    asc, csc = _bn_coeffs(ssc, sssc, float(N * P), gs, bes)
    sc = ysc * asc[None, :] + csc[None, :]

    out = _final(y1, a1, c1, sc, 8192)
    return jnp.transpose(out.reshape(N, P, C), (0, 2, 1))


# speedup vs baseline: 2.2397x; 1.5133x over previous
import functools

import jax
import jax.numpy as jnp
from jax import lax
from jax.experimental import pallas as pl
from jax.experimental.pallas import tpu as pltpu
from jax.experimental.pallas import tpu_sc as plsc

K = 16
N = 16
P = 2048
C = 64
EDGES = N * P * K
RT = 256
IMAX = 2147483647


def _knn_kernel(pts_r_ref, ptsT_ref, out_ref):
    n = pl.program_id(0)
    pr = pts_r_ref[0]
    pcT = ptsT_ref[0]
    rr = jnp.sum(pr * pr, axis=1, keepdims=True)
    rc = jnp.sum(pcT * pcT, axis=0, keepdims=True)
    dot = jnp.dot(pr.astype(jnp.bfloat16), pcT.astype(jnp.bfloat16),
                  preferred_element_type=jnp.float32)
    dist = rr - 2.0 * dot + rc
    col = lax.broadcasted_iota(jnp.int32, (RT, P), 1)
    bits = lax.bitcast_convert_type(dist, jnp.int32)
    okey = jnp.where(bits >= 0, bits, jnp.bitwise_xor(bits, jnp.int32(0x7FFFFFFF)))
    keys0 = jnp.bitwise_or(jnp.bitwise_and(okey, jnp.int32(-2048)), col)

    def body(k, carry):
        gprev, acc = carry
        m = None
        for j in range(16):
            kj = keys0[:, j * 128:(j + 1) * 128]
            t = jnp.where(kj > gprev, kj, IMAX)
            m = t if m is None else jnp.minimum(m, t)
        g = jnp.min(m, axis=1, keepdims=True)
        lane = lax.broadcasted_iota(jnp.int32, (RT, K + 1), 1)
        acc = jnp.where(lane == k, jnp.bitwise_and(g, jnp.int32(2047)), acc)
        return g, acc

    _, acc = lax.fori_loop(0, K + 1, body,
                           (jnp.full((RT, 1), -2147483648, jnp.int32),
                            jnp.zeros((RT, K + 1), jnp.int32)))
    out_ref[0] = acc[:, 1:] + n * P


def _knn(pts):
    ptsT = jnp.transpose(pts, (0, 2, 1))
    return pl.pallas_call(
        _knn_kernel,
        grid=(N, P // RT),
        in_specs=[
            pl.BlockSpec((1, RT, 2), lambda n, r: (n, r, 0)),
            pl.BlockSpec((1, 2, P), lambda n, r: (n, 0, 0)),
        ],
        out_specs=pl.BlockSpec((1, RT, K), lambda n, r: (n, r, 0)),
        out_shape=jax.ShapeDtypeStruct((N, P, K), jnp.int32),
    )(pts, ptsT)


NW = 32
CH = 1024
EPW = EDGES // NW


def _sc_gather(table, idx):
    mesh = plsc.VectorSubcoreMesh(core_axis_name="c", subcore_axis_name="s")

    @functools.partial(
        pl.kernel, mesh=mesh,
        out_type=jax.ShapeDtypeStruct((EDGES, C), jnp.float32),
        compiler_params=pltpu.CompilerParams(use_tc_tiling_on_sc=False),
        scratch_types=[
            pltpu.VMEM((CH,), jnp.int32),
            pltpu.VMEM((CH, C), jnp.float32),
            pltpu.SemaphoreType.DMA,
        ],
    )
    def k(table_hbm, idx_hbm, out_hbm, idx_v, rows_v, sem):
        wid = lax.axis_index("s") * 2 + lax.axis_index("c")
        base = wid * EPW

        def body(j, carry):
            off = base + j * CH
            pltpu.sync_copy(idx_hbm.at[pl.ds(off, CH)], idx_v)
            pltpu.async_copy(table_hbm.at[idx_v], rows_v, sem).wait()
            pltpu.sync_copy(rows_v, out_hbm.at[pl.ds(off, CH)])
            return carry

        lax.fori_loop(0, EPW // CH, body, 0)

    return k(table, idx)


def _vb(v_ref, rows):
    v = v_ref[...]
    return jnp.broadcast_to(v[:, None, :], (rows // K, K, 64)).reshape(rows, 64)


def _acc2(i, s_ref, ss_ref, s, ss):
    @pl.when(i == 0)
    def _():
        s_ref[...] = s
        ss_ref[...] = ss

    @pl.when(i > 0)
    def _():
        s_ref[...] += s
        ss_ref[...] += ss


def _stats_kernel(g_ref, v_ref, s_ref, ss_ref):
    x = _vb(v_ref, g_ref.shape[0]) - g_ref[...]
    _acc2(pl.program_id(0), s_ref, ss_ref,
          jnp.sum(x, axis=0, keepdims=True),
          jnp.sum(x * x, axis=0, keepdims=True))


def _stats(g, v, rows_per_block):
    rows = g.shape[0]
    grid = rows // rows_per_block
    s, ss = pl.pallas_call(
        _stats_kernel,
        grid=(grid,),
        in_specs=[
            pl.BlockSpec((rows_per_block, 64), lambda i: (i, 0)),
            pl.BlockSpec((rows_per_block // K, 64), lambda i: (i, 0)),
        ],
        out_specs=[
            pl.BlockSpec((1, 64), lambda i: (0, 0)),
            pl.BlockSpec((1, 64), lambda i: (0, 0)),
        ],
        out_shape=[
            jax.ShapeDtypeStruct((1, 64), jnp.float32),
            jax.ShapeDtypeStruct((1, 64), jnp.float32),
        ],
    )(g, v)
    return s[0], ss[0]


def _sc_stats_kernel(x_ref, s_ref, ss_ref):
    x = x_ref[...]
    _acc2(pl.program_id(0), s_ref, ss_ref,
          jnp.sum(x, axis=0, keepdims=True),
          jnp.sum(x * x, axis=0, keepdims=True))


def _sc_stats(x, rows_per_block):
    rows = x.shape[0]
    grid = rows // rows_per_block
    s, ss = pl.pallas_call(
        _sc_stats_kernel,
        grid=(grid,),
        in_specs=[pl.BlockSpec((rows_per_block, 64), lambda i: (i, 0))],
        out_specs=[
            pl.BlockSpec((1, 64), lambda i: (0, 0)),
            pl.BlockSpec((1, 64), lambda i: (0, 0)),
        ],
        out_shape=[
            jax.ShapeDtypeStruct((1, 64), jnp.float32),
            jax.ShapeDtypeStruct((1, 64), jnp.float32),
        ],
    )(x)
    return s[0], ss[0]


def _mid_kernel(g_ref, v_ref, a0_ref, c0_ref, w1_ref, s_ref, ss_ref):
    y0 = _vb(v_ref, g_ref.shape[0]) - g_ref[...]
    e0 = jnp.maximum(y0 * a0_ref[...] + c0_ref[...], 0.0)
    y1 = jnp.dot(e0, w1_ref[...], preferred_element_type=jnp.float32)
    _acc2(pl.program_id(0), s_ref, ss_ref,
          jnp.sum(y1, axis=0, keepdims=True),
          jnp.sum(y1 * y1, axis=0, keepdims=True))


def _mid(g, v, a0, c0, w1t, rows_per_block):
    rows = g.shape[0]
    grid = rows // rows_per_block
    s, ss = pl.pallas_call(
        _mid_kernel,
        grid=(grid,),
        in_specs=[
            pl.BlockSpec((rows_per_block, 64), lambda i: (i, 0)),
            pl.BlockSpec((rows_per_block // K, 64), lambda i: (i, 0)),
            pl.BlockSpec((1, 64), lambda i: (0, 0)),
            pl.BlockSpec((1, 64), lambda i: (0, 0)),
            pl.BlockSpec((64, 64), lambda i: (0, 0)),
        ],
        out_specs=[
            pl.BlockSpec((1, 64), lambda i: (0, 0)),
            pl.BlockSpec((1, 64), lambda i: (0, 0)),
        ],
        out_shape=[
            jax.ShapeDtypeStruct((1, 64), jnp.float32),
            jax.ShapeDtypeStruct((1, 64), jnp.float32),
        ],
    )(g, v, a0[None, :], c0[None, :], w1t)
    return s[0], ss[0]


def _final_kernel(g_ref, v_ref, a0_ref, c0_ref, w1_ref, a1_ref, c1_ref,
                  sc_ref, out_ref):
    rows = g_ref.shape[0]
    y0 = _vb(v_ref, rows) - g_ref[...]
    e0 = jnp.maximum(y0 * a0_ref[...] + c0_ref[...], 0.0)
    y1 = jnp.dot(e0, w1_ref[...], preferred_element_type=jnp.float32)
    e1 = jnp.maximum(y1 * a1_ref[...] + c1_ref[...], 0.0)
    h = jnp.mean(e1.reshape(rows // K, K, 64), axis=1)
    out_ref[...] = jnp.maximum(h + sc_ref[...], 0.0)


def _final(g, v, a0, c0, w1t, a1, c1, sc, rows_per_block):
    rows = g.shape[0]
    grid = rows // rows_per_block
    out = pl.pallas_call(
        _final_kernel,
        grid=(grid,),
        in_specs=[
            pl.BlockSpec((rows_per_block, 64), lambda i: (i, 0)),
            pl.BlockSpec((rows_per_block // K, 64), lambda i: (i, 0)),
            pl.BlockSpec((1, 64), lambda i: (0, 0)),
            pl.BlockSpec((1, 64), lambda i: (0, 0)),
            pl.BlockSpec((64, 64), lambda i: (0, 0)),
            pl.BlockSpec((1, 64), lambda i: (0, 0)),
            pl.BlockSpec((1, 64), lambda i: (0, 0)),
            pl.BlockSpec((rows_per_block // K, 64), lambda i: (i, 0)),
        ],
        out_specs=pl.BlockSpec((rows_per_block // K, 64), lambda i: (i, 0)),
        out_shape=jax.ShapeDtypeStruct((rows // K, 64), jnp.float32),
    )(g, v, a0[None, :], c0[None, :], w1t, a1[None, :], c1[None, :], sc)
    return out


def _bn_coeffs(s, ss, count, g, be):
    m = s / count
    v = ss / count - m * m
    a = g * lax.rsqrt(v + 1e-5)
    c = be - m * a
    return a, c


def kernel(features, W0, b0, g0, be0, W1, b1, g1, be1, Ws, bs, gs, bes):
    pts = features[:, :, 0:2]
    X = features[:, :, 2:]

    gidx = _knn(pts).reshape(EDGES)

    W0a = W0[:, :C]
    W0b = W0[:, C:]
    U = jnp.einsum('npc,oc->npo', X, W0b).reshape(N * P, C)
    V = jnp.einsum('npc,oc->npo', X, W0a + W0b) + b0[None, None, :]

    G = _sc_gather(U, gidx)
    Vf = V.reshape(N * P, C)

    s0, ss0 = _stats(G, Vf, 8192)
    a0, c0 = _bn_coeffs(s0, ss0, float(EDGES), g0, be0)
    s1, ss1 = _mid(G, Vf, a0, c0, W1.T, 8192)
    a1, c1 = _bn_coeffs(s1, ss1, float(EDGES), g1, be1)

    ysc = (jnp.einsum('npc,oc->npo', X, Ws) + bs[None, None, :]).reshape(N * P, C)
    ssc, sssc = _sc_stats(ysc, 4096)
    asc, csc = _bn_coeffs(ssc, sssc, float(N * P), gs, bes)
    sc = ysc * asc[None, :] + csc[None, :]

    out = _final(G, Vf, a0, c0, W1.T, a1, c1, sc, 8192)
    return jnp.transpose(out.reshape(N, P, C), (0, 2, 1))
